# baseline (speedup 1.0000x reference)
import functools

import jax
import jax.numpy as jnp
from jax import lax
from jax.experimental import pallas as pl
from jax.experimental.pallas import tpu as pltpu
from jax.experimental.pallas import tpu_sc as plsc

_NC = 2
_NS = 16
_NW = _NC * _NS

_BATCH = 16384
_HIST = 50
_D = 32
_V = 1000000
_B = _BATCH * _HIST
_BPW = _B // _NW
_NBUF = 4
_C = 640
_NCHUNK = _BPW // _C

_TCOL = _V // 128
_COLS_PER_W = 245


def _mesh():
    return plsc.VectorSubcoreMesh(
        core_axis_name="c", subcore_axis_name="s",
        num_cores=_NC, num_subcores=_NS,
    )


def _wid():
    return lax.axis_index("s") * _NC + lax.axis_index("c")


def _relayout_body(tT_hbm, tail_hbm, trm_hbm, stage_v, trans_v, *, last_col):
    wid = _wid()
    rows16 = lax.iota(jnp.int32, 16)

    def transpose_col(lane_shift, l0, nl):
        @pl.loop(l0, l0 + nl, unroll=8)
        def _row(l):
            src_l = l + lane_shift
            for d0 in (0, 16):
                v = plsc.load_gather(
                    stage_v, [d0 + rows16, jnp.full((16,), src_l, jnp.int32)]
                )
                trans_v[pl.ds(l * 32 + d0, 16)] = v

    @pl.loop(0, _COLS_PER_W)
    def _col(g):
        t = wid + _NW * g

        @pl.when(t < _TCOL)
        def _():
            off = pl.multiple_of(t * 128, 128)
            pltpu.sync_copy(tT_hbm.at[:, pl.ds(off, 128)], stage_v)
            transpose_col(0, 0, 128)
            pltpu.sync_copy(trans_v, trm_hbm.at[pl.ds(t * 4096, 4096)])

    if last_col:
        @pl.when(wid == (_TCOL % _NW))
        def _tail():
            pltpu.sync_copy(tail_hbm,
                            trm_hbm.at[pl.ds(_TCOL * 128 * _D, 64 * _D)])


def _make_relayout():
    return functools.partial(
        pl.kernel,
        out_type=jax.ShapeDtypeStruct((_V * _D,), jnp.float32),
        mesh=_mesh(),
        scratch_types=[
            pltpu.VMEM((_D, 128), jnp.float32),
            pltpu.VMEM((128 * _D,), jnp.float32),
        ],
        compiler_params=pltpu.CompilerParams(needs_layout_passes=False),
    )(functools.partial(_relayout_body, last_col=True))


def _gather_body(x_hbm, table2d, out_hbm, idx_v, rows_v, *sems):
    gsems, osems = sems[:_NBUF], sems[_NBUF:]
    wid = _wid()
    base = wid * _BPW
    pltpu.sync_copy(x_hbm.at[pl.ds(base, _BPW)], idx_v)

    def start_gather(c, b):
        off = pl.multiple_of(c * _C, _C)
        pltpu.async_copy(
            table2d.at[idx_v.at[pl.ds(off, _C)]], rows_v.at[b], gsems[b]
        )

    def start_write(c, b):
        off = pl.multiple_of(c * _C, _C)
        pltpu.async_copy(rows_v.at[b], out_hbm.at[pl.ds(base + off, _C)],
                         osems[b])

    def drain_gather(b):
        pltpu.make_async_copy(
            table2d.at[pl.ds(0, _C)], rows_v.at[b], gsems[b]
        ).wait()

    def drain_write(b):
        pltpu.make_async_copy(
            rows_v.at[b], out_hbm.at[pl.ds(base, _C)], osems[b]
        ).wait()

    for b in range(_NBUF):
        start_gather(b, b)

    @pl.loop(0, _NCHUNK - _NBUF, step=_NBUF)
    def _round(g):
        for b in range(_NBUF):
            c = g + b
            drain_gather(b)
            start_write(c, b)
            drain_write(b)
            start_gather(c + _NBUF, b)

    for b in range(_NBUF):
        drain_gather(b)
        start_write(_NCHUNK - _NBUF + b, b)
    for b in range(_NBUF):
        drain_write(b)


def _make_gather():
    return functools.partial(
        pl.kernel,
        out_type=jax.ShapeDtypeStruct((_B, _D), jnp.float32),
        mesh=_mesh(),
        scratch_types=(
            [pltpu.VMEM((_BPW,), jnp.int32),
             pltpu.VMEM((_NBUF, _C, _D), jnp.float32)]
            + [pltpu.SemaphoreType.DMA] * (2 * _NBUF)
        ),
        compiler_params=pltpu.CompilerParams(use_tc_tiling_on_sc=False),
    )(_gather_body)


def kernel(x, table):
    xf = x.reshape(_B)
    tail = table[_TCOL * 128:].reshape(64 * _D)
    trm = _make_relayout()(table.T, tail)
    out = _make_gather()(xf, trm.reshape(_V, _D))
    return out.reshape(_BATCH, _HIST, _D)


# speedup vs baseline: 1.2770x; 1.2770x over previous
import functools

import jax
import jax.numpy as jnp
from jax import lax
from jax.experimental import pallas as pl
from jax.experimental.pallas import tpu as pltpu
from jax.experimental.pallas import tpu_sc as plsc

_NC = 2
_NS = 16
_NW = _NC * _NS

_BATCH = 16384
_HIST = 50
_D = 32
_V = 1000000
_B = _BATCH * _HIST
_BPW = _B // _NW
_NBUF = 4
_C = 640
_NCHUNK = _BPW // _C

_TCOL = _V // 128
_COLS_PER_W = 245


def _mesh():
    return plsc.VectorSubcoreMesh(
        core_axis_name="c", subcore_axis_name="s",
        num_cores=_NC, num_subcores=_NS,
    )


def _wid():
    return lax.axis_index("s") * _NC + lax.axis_index("c")


def _relayout_body(tT_hbm, tail_hbm, trm_hbm, stage_v, trans_v, *, last_col):
    wid = _wid()
    rows16 = lax.iota(jnp.int32, 16)

    def transpose_col(lane_shift, l0, nl):
        @pl.loop(l0, l0 + nl, unroll=8)
        def _row(l):
            src_l = l + lane_shift
            for d0 in (0, 16):
                v = plsc.load_gather(
                    stage_v, [d0 + rows16, jnp.full((16,), src_l, jnp.int32)]
                )
                trans_v[pl.ds(l * 32 + d0, 16)] = v

    @pl.loop(0, _COLS_PER_W)
    def _col(g):
        t = wid + _NW * g

        @pl.when(t < _TCOL)
        def _():
            off = pl.multiple_of(t * 128, 128)
            pltpu.sync_copy(tT_hbm.at[:, pl.ds(off, 128)], stage_v)
            transpose_col(0, 0, 128)
            pltpu.sync_copy(trans_v, trm_hbm.at[pl.ds(t * 4096, 4096)])

    if last_col:
        @pl.when(wid == (_TCOL % _NW))
        def _tail():
            pltpu.sync_copy(tail_hbm,
                            trm_hbm.at[pl.ds(_TCOL * 128 * _D, 64 * _D)])


def _make_relayout():
    return functools.partial(
        pl.kernel,
        out_type=jax.ShapeDtypeStruct((_V * _D,), jnp.float32),
        mesh=_mesh(),
        scratch_types=[
            pltpu.VMEM((_D, 128), jnp.float32),
            pltpu.VMEM((128 * _D,), jnp.float32),
        ],
        compiler_params=pltpu.CompilerParams(needs_layout_passes=False),
    )(functools.partial(_relayout_body, last_col=True))


_Q = _V // 4
_JPW = 4
_NPAIR = _HIST * _JPW


def _lookup_body(x_hbm, t4_hbm, out_hbm, x_v, idx_b, rem_b, g_v, so_v, *sems):
    gsems, osems = sems[:2], sems[2:]
    wid = _wid()
    iota = lax.iota(jnp.int32, 16)
    iota50 = iota * 50
    pltpu.sync_copy(x_hbm.at[pl.ds(wid * _BPW, _BPW)], x_v)

    def fire(p, s):
        jj = p // _HIST
        h = p % _HIST
        for l0 in range(8):
            f_vec = iota50 + (jj * 6400 + l0 * 800 + h)
            xq = plsc.load_gather(x_v, [f_vec])
            idx_b[s, pl.ds(l0 * 16, 16)] = xq >> 2
            rem_b[s, pl.ds(l0 * 16, 16)] = (xq & 3) << 5
        pltpu.async_copy(t4_hbm.at[idx_b.at[s]], g_v.at[s], gsems[s])

    def drain_gather(s):
        pltpu.make_async_copy(t4_hbm.at[pl.ds(0, 128)], g_v.at[s],
                              gsems[s]).wait()

    def drain_out(s):
        pltpu.make_async_copy(so_v.at[s], out_hbm.at[0, :, pl.ds(0, 128)],
                              osems[s]).wait()

    def consume(p, s):
        jj = p // _HIST
        h = p % _HIST
        for l0 in range(8):
            rows = l0 * 16 + iota
            remb = rem_b[s, pl.ds(l0 * 16, 16)]
            for d in range(_D):
                v = plsc.load_gather(g_v.at[s], [rows, remb + d])
                so_v[s, d, pl.ds(l0 * 16, 16)] = v
        off = pl.multiple_of((wid * _JPW + jj) * 128, 128)
        pltpu.async_copy(so_v.at[s], out_hbm.at[h, :, pl.ds(off, 128)],
                         osems[s])

    fire(0, 0)
    fire(1, 1)

    @pl.loop(0, _NPAIR - 2, step=2)
    def _pair(pp):
        for s in (0, 1):
            p = pp + s
            drain_gather(s)

            @pl.when(pp > 0)
            def _():
                drain_out(s)

            consume(p, s)
            fire(p + 2, s)

    for s in (0, 1):
        drain_gather(s)
        drain_out(s)
        consume(_NPAIR - 2 + s, s)
    for s in (0, 1):
        drain_out(s)


def _make_lookup():
    return functools.partial(
        pl.kernel,
        out_type=jax.ShapeDtypeStruct((_HIST, _D, _BATCH), jnp.float32),
        mesh=_mesh(),
        scratch_types=(
            [pltpu.VMEM((_BPW,), jnp.int32),
             pltpu.VMEM((2, 128), jnp.int32),
             pltpu.VMEM((2, 128), jnp.int32),
             pltpu.VMEM((2, 128, 128), jnp.float32),
             pltpu.VMEM((2, _D, 128), jnp.float32)]
            + [pltpu.SemaphoreType.DMA] * 4
        ),
        compiler_params=pltpu.CompilerParams(needs_layout_passes=False),
    )(_lookup_body)


def kernel(x, table):
    xf = x.reshape(_B)
    tail = table[_TCOL * 128:].reshape(64 * _D)
    trm = _make_relayout()(table.T, tail)
    out = _make_lookup()(xf, trm.reshape(_Q, 128))
    return out.transpose(2, 0, 1)


# speedup vs baseline: 1.6594x; 1.2995x over previous
import functools

import jax
import jax.numpy as jnp
from jax import lax
from jax.experimental import pallas as pl
from jax.experimental.pallas import tpu as pltpu
from jax.experimental.pallas import tpu_sc as plsc

_NC = 2
_NS = 16
_NW = _NC * _NS

_BATCH = 16384
_HIST = 50
_D = 32
_V = 1000000
_B = _BATCH * _HIST
_BPW = _B // _NW
_NBUF = 4
_C = 640
_NCHUNK = _BPW // _C

_TCOL = _V // 128
_COLS_PER_W = 245


def _mesh():
    return plsc.VectorSubcoreMesh(
        core_axis_name="c", subcore_axis_name="s",
        num_cores=_NC, num_subcores=_NS,
    )


def _wid():
    return lax.axis_index("s") * _NC + lax.axis_index("c")


_KCOL = _TCOL // _NW
_RB = 2


def _relayout_body(tT_hbm, tail_hbm, trm_hbm, stage0, stage1, trans0,
                   trans1, *sems):
    stages = [stage0, stage1]
    transs = [trans0, trans1]
    isems, osems = sems[:_RB], sems[_RB:]
    wid = _wid()
    iota32 = lax.iota(jnp.int32, 16) * _D

    def start_in(t, b):
        off = pl.multiple_of(t * 128, 128)
        pltpu.async_copy(tT_hbm.at[:, pl.ds(off, 128)], stages[b], isems[b])

    def drain_in(b):
        pltpu.make_async_copy(tT_hbm.at[:, pl.ds(0, 128)], stages[b],
                              isems[b]).wait()

    def start_out(t, b):
        pltpu.async_copy(transs[b], trm_hbm.at[pl.ds(t * 4096, 4096)],
                         osems[b])

    def drain_out(b):
        pltpu.make_async_copy(transs[b], trm_hbm.at[pl.ds(0, 4096)],
                              osems[b]).wait()

    bases = [iota32 + l0 * 16 * _D for l0 in range(8)]

    def transpose(b):
        for d in range(_D):
            for l0 in range(8):
                v = stages[b][d, pl.ds(l0 * 16, 16)]
                plsc.store_scatter(transs[b], [bases[l0] + d], v)

    for b in range(_RB):
        start_in(wid + b * _NW, b)

    @pl.loop(0, _KCOL, step=_RB)
    def _col(g):
        for b in range(_RB):
            c = g + b
            t = wid + c * _NW
            drain_in(b)

            @pl.when(g > 0)
            def _():
                drain_out(b)

            transpose(b)
            start_out(t, b)

            @pl.when(c + _RB < _KCOL)
            def _():
                start_in(wid + (c + _RB) * _NW, b)

    for b in range(_RB):
        drain_out(b)

    @pl.when(wid < _TCOL - _KCOL * _NW)
    def _extra():
        t = wid + _KCOL * _NW
        off = pl.multiple_of(t * 128, 128)
        pltpu.sync_copy(tT_hbm.at[:, pl.ds(off, 128)], stages[0])
        transpose(0)
        pltpu.sync_copy(transs[0], trm_hbm.at[pl.ds(t * 4096, 4096)])

    @pl.when(wid == (_TCOL % _NW))
    def _tail():
        pltpu.sync_copy(tail_hbm,
                        trm_hbm.at[pl.ds(_TCOL * 128 * _D, 64 * _D)])


def _make_relayout():
    return functools.partial(
        pl.kernel,
        out_type=jax.ShapeDtypeStruct((_V * _D,), jnp.float32),
        mesh=_mesh(),
        scratch_types=(
            [pltpu.VMEM((_D, 128), jnp.float32)] * _RB
            + [pltpu.VMEM((128 * _D,), jnp.float32)] * _RB
            + [pltpu.SemaphoreType.DMA] * (2 * _RB)
        ),
        compiler_params=pltpu.CompilerParams(needs_layout_passes=False),
    )(_relayout_body)


_Q = _V // 4
_JPW = 4
_NPAIR = _HIST * _JPW


_LB = 2


def _lookup_body(x_hbm, t4_hbm, out_hbm, x_v, idx_b, rem_b, g_v, so_v, *sems):
    gsems, osems = sems[:_LB], sems[_LB:]
    wid = _wid()
    iota = lax.iota(jnp.int32, 16)
    iota50 = iota * 50
    pltpu.sync_copy(x_hbm.at[pl.ds(wid * _BPW, _BPW)], x_v)

    def fire(p, s):
        jj = p // _HIST
        h = p % _HIST
        for l0 in range(8):
            f_vec = iota50 + (jj * 6400 + l0 * 800 + h)
            xq = plsc.load_gather(x_v, [f_vec])
            idx_b[s, pl.ds(l0 * 16, 16)] = xq >> 2
            rem_b[s, pl.ds(l0 * 16, 16)] = (xq & 3) << 5
        pltpu.async_copy(t4_hbm.at[idx_b.at[s]], g_v.at[s], gsems[s])

    def drain_gather(s):
        pltpu.make_async_copy(t4_hbm.at[pl.ds(0, 128)], g_v.at[s],
                              gsems[s]).wait()

    def drain_out(s):
        pltpu.make_async_copy(so_v.at[s], out_hbm.at[0, :, pl.ds(0, 128)],
                              osems[s]).wait()

    def consume(p, s):
        jj = p // _HIST
        h = p % _HIST
        rows = [l0 * 16 + iota for l0 in range(8)]
        rembs = [rem_b[s, pl.ds(l0 * 16, 16)] for l0 in range(8)]
        for d in range(_D):
            for l0 in range(8):
                v = plsc.load_gather(g_v.at[s], [rows[l0], rembs[l0] + d])
                so_v[s, d, pl.ds(l0 * 16, 16)] = v
        off = pl.multiple_of((wid * _JPW + jj) * 128, 128)
        pltpu.async_copy(so_v.at[s], out_hbm.at[h, :, pl.ds(off, 128)],
                         osems[s])

    for s in range(_LB):
        fire(s, s)

    @pl.loop(0, _NPAIR - _LB, step=_LB)
    def _pair(pp):
        for s in range(_LB):
            p = pp + s
            drain_gather(s)

            @pl.when(pp > 0)
            def _():
                drain_out(s)

            consume(p, s)
            fire(p + _LB, s)

    for s in range(_LB):
        drain_gather(s)
        drain_out(s)
        consume(_NPAIR - _LB + s, s)
    for s in range(_LB):
        drain_out(s)


def _make_lookup():
    return functools.partial(
        pl.kernel,
        out_type=jax.ShapeDtypeStruct((_HIST, _D, _BATCH), jnp.float32),
        mesh=_mesh(),
        scratch_types=(
            [pltpu.VMEM((_BPW,), jnp.int32),
             pltpu.VMEM((_LB, 128), jnp.int32),
             pltpu.VMEM((_LB, 128), jnp.int32),
             pltpu.VMEM((_LB, 128, 128), jnp.float32),
             pltpu.VMEM((_LB, _D, 128), jnp.float32)]
            + [pltpu.SemaphoreType.DMA] * (2 * _LB)
        ),
        compiler_params=pltpu.CompilerParams(needs_layout_passes=False),
    )(_lookup_body)


def kernel(x, table):
    xf = x.reshape(_B)
    tail = table[_TCOL * 128:].reshape(64 * _D)
    trm = _make_relayout()(table.T, tail)
    out = _make_lookup()(xf, trm.reshape(_Q, 128))
    return out.transpose(2, 0, 1)


# speedup vs baseline: 1.6626x; 1.0019x over previous
import functools

import jax
import jax.numpy as jnp
from jax import lax
from jax.experimental import pallas as pl
from jax.experimental.pallas import tpu as pltpu
from jax.experimental.pallas import tpu_sc as plsc

_NC = 2
_NS = 16
_NW = _NC * _NS

_BATCH = 16384
_HIST = 50
_D = 32
_V = 1000000
_B = _BATCH * _HIST
_BPW = _B // _NW
_NBUF = 4
_C = 640
_NCHUNK = _BPW // _C

_TCOL = _V // 128
_COLS_PER_W = 245


def _mesh():
    return plsc.VectorSubcoreMesh(
        core_axis_name="c", subcore_axis_name="s",
        num_cores=_NC, num_subcores=_NS,
    )


def _wid():
    return lax.axis_index("s") * _NC + lax.axis_index("c")


_KCOL = _TCOL // _NW
_RB = 2


def _relayout_body(tT_hbm, tail_hbm, trm_hbm, stage0, stage1, trans0,
                   trans1, *sems):
    stages = [stage0, stage1]
    transs = [trans0, trans1]
    isems, osems = sems[:_RB], sems[_RB:]
    wid = _wid()
    iota32 = lax.iota(jnp.int32, 16) * _D

    def start_in(t, b):
        off = pl.multiple_of(t * 128, 128)
        pltpu.async_copy(tT_hbm.at[:, pl.ds(off, 128)], stages[b], isems[b])

    def drain_in(b):
        pltpu.make_async_copy(tT_hbm.at[:, pl.ds(0, 128)], stages[b],
                              isems[b]).wait()

    def start_out(t, b):
        pltpu.async_copy(transs[b], trm_hbm.at[pl.ds(t * 4096, 4096)],
                         osems[b])

    def drain_out(b):
        pltpu.make_async_copy(transs[b], trm_hbm.at[pl.ds(0, 4096)],
                              osems[b]).wait()

    bases = [iota32 + l0 * 16 * _D for l0 in range(8)]

    def transpose(b):
        for d in range(_D):
            vs = [stages[b][d, pl.ds(l0 * 16, 16)] for l0 in range(8)]
            for l0 in range(8):
                plsc.store_scatter(transs[b], [bases[l0] + d], vs[l0])

    for b in range(_RB):
        start_in(wid + b * _NW, b)

    @pl.loop(0, _KCOL, step=_RB)
    def _col(g):
        for b in range(_RB):
            c = g + b
            t = wid + c * _NW
            drain_in(b)

            @pl.when(g > 0)
            def _():
                drain_out(b)

            transpose(b)
            start_out(t, b)

            @pl.when(c + _RB < _KCOL)
            def _():
                start_in(wid + (c + _RB) * _NW, b)

    for b in range(_RB):
        drain_out(b)

    @pl.when(wid < _TCOL - _KCOL * _NW)
    def _extra():
        t = wid + _KCOL * _NW
        off = pl.multiple_of(t * 128, 128)
        pltpu.sync_copy(tT_hbm.at[:, pl.ds(off, 128)], stages[0])
        transpose(0)
        pltpu.sync_copy(transs[0], trm_hbm.at[pl.ds(t * 4096, 4096)])

    @pl.when(wid == (_TCOL % _NW))
    def _tail():
        pltpu.sync_copy(tail_hbm,
                        trm_hbm.at[pl.ds(_TCOL * 128 * _D, 64 * _D)])


def _make_relayout():
    return functools.partial(
        pl.kernel,
        out_type=jax.ShapeDtypeStruct((_V * _D,), jnp.float32),
        mesh=_mesh(),
        scratch_types=(
            [pltpu.VMEM((_D, 128), jnp.float32)] * _RB
            + [pltpu.VMEM((128 * _D,), jnp.float32)] * _RB
            + [pltpu.SemaphoreType.DMA] * (2 * _RB)
        ),
        compiler_params=pltpu.CompilerParams(needs_layout_passes=False),
    )(_relayout_body)


_Q = _V // 4
_JPW = 4
_NPAIR = _HIST * _JPW


_LB = 2


def _lookup_body(x_hbm, t4_hbm, out_hbm, x_v, idx_b, rem_b, g_v, so_v, *sems):
    gsems, osems = sems[:_LB], sems[_LB:]
    wid = _wid()
    iota = lax.iota(jnp.int32, 16)
    iota50 = iota * 50
    pltpu.sync_copy(x_hbm.at[pl.ds(wid * _BPW, _BPW)], x_v)

    def fire(p, s):
        jj = p // _HIST
        h = p % _HIST
        for l0 in range(8):
            f_vec = iota50 + (jj * 6400 + l0 * 800 + h)
            xq = plsc.load_gather(x_v, [f_vec])
            idx_b[s, pl.ds(l0 * 16, 16)] = xq >> 2
            rem_b[s, pl.ds(l0 * 16, 16)] = (xq & 3) << 5
        pltpu.async_copy(t4_hbm.at[idx_b.at[s]], g_v.at[s], gsems[s])

    def drain_gather(s):
        pltpu.make_async_copy(t4_hbm.at[pl.ds(0, 128)], g_v.at[s],
                              gsems[s]).wait()

    def drain_out(s):
        pltpu.make_async_copy(so_v.at[s], out_hbm.at[0, :, pl.ds(0, 128)],
                              osems[s]).wait()

    def consume(p, s):
        jj = p // _HIST
        h = p % _HIST
        rows = [l0 * 16 + iota for l0 in range(8)]
        rembs = [rem_b[s, pl.ds(l0 * 16, 16)] for l0 in range(8)]
        for d in range(_D):
            for l0 in range(8):
                v = plsc.load_gather(g_v.at[s], [rows[l0], rembs[l0] + d])
                so_v[s, d, pl.ds(l0 * 16, 16)] = v
        off = pl.multiple_of((wid * _JPW + jj) * 128, 128)
        pltpu.async_copy(so_v.at[s], out_hbm.at[h, :, pl.ds(off, 128)],
                         osems[s])

    for s in range(_LB):
        fire(s, s)

    @pl.loop(0, _NPAIR - _LB, step=_LB)
    def _pair(pp):
        for s in range(_LB):
            p = pp + s
            drain_gather(s)

            @pl.when(pp > 0)
            def _():
                drain_out(s)

            consume(p, s)
            fire(p + _LB, s)

    for s in range(_LB):
        drain_gather(s)
        drain_out(s)
        consume(_NPAIR - _LB + s, s)
    for s in range(_LB):
        drain_out(s)


def _make_lookup():
    return functools.partial(
        pl.kernel,
        out_type=jax.ShapeDtypeStruct((_HIST, _D, _BATCH), jnp.float32),
        mesh=_mesh(),
        scratch_types=(
            [pltpu.VMEM((_BPW,), jnp.int32),
             pltpu.VMEM((_LB, 128), jnp.int32),
             pltpu.VMEM((_LB, 128), jnp.int32),
             pltpu.VMEM((_LB, 128, 128), jnp.float32),
             pltpu.VMEM((_LB, _D, 128), jnp.float32)]
            + [pltpu.SemaphoreType.DMA] * (2 * _LB)
        ),
        compiler_params=pltpu.CompilerParams(needs_layout_passes=False),
    )(_lookup_body)


def kernel(x, table):
    xf = x.reshape(_B)
    tail = table[_TCOL * 128:].reshape(64 * _D)
    trm = _make_relayout()(table.T, tail)
    out = _make_lookup()(xf, trm.reshape(_Q, 128))
    return out.transpose(2, 0, 1)


# speedup vs baseline: 2.4257x; 1.4590x over previous
import functools

import jax
import jax.numpy as jnp
from jax import lax
from jax.experimental import pallas as pl
from jax.experimental.pallas import tpu as pltpu
from jax.experimental.pallas import tpu_sc as plsc

_NC = 2
_NS = 16
_NW = _NC * _NS

_BATCH = 16384
_HIST = 50
_D = 32
_V = 1000000
_B = _BATCH * _HIST
_BPW = _B // _NW
_NBUF = 4
_C = 640
_NCHUNK = _BPW // _C

_TCOL = _V // 128
_COLS_PER_W = 245


def _mesh():
    return plsc.VectorSubcoreMesh(
        core_axis_name="c", subcore_axis_name="s",
        num_cores=_NC, num_subcores=_NS,
    )


def _wid():
    return lax.axis_index("s") * _NC + lax.axis_index("c")


_KCOL = _TCOL // _NW
_RB = 2


def _relayout_body(tT_hbm, tail_hbm, trm_hbm, stage0, stage1, trans0,
                   trans1, *sems):
    stages = [stage0, stage1]
    transs = [trans0, trans1]
    isems, osems = sems[:_RB], sems[_RB:]
    wid = _wid()
    iota32 = lax.iota(jnp.int32, 16) * _D

    def start_in(t, b):
        off = pl.multiple_of(t * 128, 128)
        pltpu.async_copy(tT_hbm.at[:, pl.ds(off, 128)], stages[b], isems[b])

    def drain_in(b):
        pltpu.make_async_copy(tT_hbm.at[:, pl.ds(0, 128)], stages[b],
                              isems[b]).wait()

    def start_out(t, b):
        pltpu.async_copy(transs[b], trm_hbm.at[pl.ds(t * 4096, 4096)],
                         osems[b])

    def drain_out(b):
        pltpu.make_async_copy(transs[b], trm_hbm.at[pl.ds(0, 4096)],
                              osems[b]).wait()

    bases = [iota32 + l0 * 16 * _D for l0 in range(8)]

    def transpose(b):
        for d in range(_D):
            vs = [stages[b][d, pl.ds(l0 * 16, 16)] for l0 in range(8)]
            for l0 in range(8):
                plsc.store_scatter(transs[b], [bases[l0] + d], vs[l0])

    for b in range(_RB):
        start_in(wid + b * _NW, b)

    @pl.loop(0, _KCOL, step=_RB)
    def _col(g):
        for b in range(_RB):
            c = g + b
            t = wid + c * _NW
            drain_in(b)

            @pl.when(g > 0)
            def _():
                drain_out(b)

            transpose(b)
            start_out(t, b)

            @pl.when(c + _RB < _KCOL)
            def _():
                start_in(wid + (c + _RB) * _NW, b)

    for b in range(_RB):
        drain_out(b)

    @pl.when(wid < _TCOL - _KCOL * _NW)
    def _extra():
        t = wid + _KCOL * _NW
        off = pl.multiple_of(t * 128, 128)
        pltpu.sync_copy(tT_hbm.at[:, pl.ds(off, 128)], stages[0])
        transpose(0)
        pltpu.sync_copy(transs[0], trm_hbm.at[pl.ds(t * 4096, 4096)])

    @pl.when(wid == (_TCOL % _NW))
    def _tail():
        pltpu.sync_copy(tail_hbm,
                        trm_hbm.at[pl.ds(_TCOL * 128 * _D, 64 * _D)])


def _make_relayout():
    return functools.partial(
        pl.kernel,
        out_type=jax.ShapeDtypeStruct((_V * _D,), jnp.float32),
        mesh=_mesh(),
        scratch_types=(
            [pltpu.VMEM((_D, 128), jnp.float32)] * _RB
            + [pltpu.VMEM((128 * _D,), jnp.float32)] * _RB
            + [pltpu.SemaphoreType.DMA] * (2 * _RB)
        ),
        compiler_params=pltpu.CompilerParams(needs_layout_passes=False),
    )(_relayout_body)


_Q = _V // 4
_JPW = 4
_NPAIR = _HIST * _JPW


_LB = 2


def _lookup_body(x_hbm, t4_hbm, out_hbm, x_v, idx_b, rem_b, g_v, so_v, *sems):
    gsems, osems = sems[:_LB], sems[_LB:]
    wid = _wid()
    iota = lax.iota(jnp.int32, 16)
    iota50 = iota * 50
    pltpu.sync_copy(x_hbm.at[pl.ds(wid * _BPW, _BPW)], x_v)

    def fire(p, s):
        jj = p // _HIST
        h = p % _HIST
        for l0 in range(8):
            f_vec = iota50 + (jj * 6400 + l0 * 800 + h)
            xq = plsc.load_gather(x_v, [f_vec])
            idx_b[s, pl.ds(l0 * 16, 16)] = xq >> 2
            rem_b[s, pl.ds(l0 * 16, 16)] = (xq & 3) << 5
        pltpu.async_copy(t4_hbm.at[idx_b.at[s]], g_v.at[s], gsems[s])

    def drain_gather(s):
        pltpu.make_async_copy(t4_hbm.at[pl.ds(0, 128)], g_v.at[s],
                              gsems[s]).wait()

    def drain_out(s):
        pltpu.make_async_copy(so_v.at[s], out_hbm.at[0, :, pl.ds(0, 128)],
                              osems[s]).wait()

    def consume(p, s):
        jj = p // _HIST
        h = p % _HIST
        rows = [l0 * 16 + iota for l0 in range(8)]
        rembs = [rem_b[s, pl.ds(l0 * 16, 16)] for l0 in range(8)]
        @pl.loop(0, 16)
        def _diag(k):
            dk = (iota + k) & 15
            for d0 in (0, 16):
                dcol = d0 + dk
                for l0 in range(8):
                    v = plsc.load_gather(g_v.at[s],
                                         [rows[l0], rembs[l0] + dcol])
                    plsc.store_scatter(so_v.at[s], [dcol, rows[l0]], v)
        off = pl.multiple_of((wid * _JPW + jj) * 128, 128)
        pltpu.async_copy(so_v.at[s], out_hbm.at[h, :, pl.ds(off, 128)],
                         osems[s])

    for s in range(_LB):
        fire(s, s)

    @pl.loop(0, _NPAIR - _LB, step=_LB)
    def _pair(pp):
        for s in range(_LB):
            p = pp + s
            drain_gather(s)

            @pl.when(pp > 0)
            def _():
                drain_out(s)

            consume(p, s)
            fire(p + _LB, s)

    for s in range(_LB):
        drain_gather(s)
        drain_out(s)
        consume(_NPAIR - _LB + s, s)
    for s in range(_LB):
        drain_out(s)


def _make_lookup():
    return functools.partial(
        pl.kernel,
        out_type=jax.ShapeDtypeStruct((_HIST, _D, _BATCH), jnp.float32),
        mesh=_mesh(),
        scratch_types=(
            [pltpu.VMEM((_BPW,), jnp.int32),
             pltpu.VMEM((_LB, 128), jnp.int32),
             pltpu.VMEM((_LB, 128), jnp.int32),
             pltpu.VMEM((_LB, 128, 128), jnp.float32),
             pltpu.VMEM((_LB, _D, 128), jnp.float32)]
            + [pltpu.SemaphoreType.DMA] * (2 * _LB)
        ),
        compiler_params=pltpu.CompilerParams(needs_layout_passes=False),
    )(_lookup_body)


def kernel(x, table):
    xf = x.reshape(_B)
    tail = table[_TCOL * 128:].reshape(64 * _D)
    trm = _make_relayout()(table.T, tail)
    out = _make_lookup()(xf, trm.reshape(_Q, 128))
    return out.transpose(2, 0, 1)


# speedup vs baseline: 4.0019x; 1.6497x over previous
import functools

import jax
import jax.numpy as jnp
from jax import lax
from jax.experimental import pallas as pl
from jax.experimental.pallas import tpu as pltpu
from jax.experimental.pallas import tpu_sc as plsc

_NC = 2
_NS = 16
_NW = _NC * _NS

_BATCH = 16384
_HIST = 50
_D = 32
_V = 1000000
_B = _BATCH * _HIST
_BPW = _B // _NW
_NBUF = 4
_C = 640
_NCHUNK = _BPW // _C

_TCOL = _V // 128
_COLS_PER_W = 245


def _mesh():
    return plsc.VectorSubcoreMesh(
        core_axis_name="c", subcore_axis_name="s",
        num_cores=_NC, num_subcores=_NS,
    )


def _wid():
    return lax.axis_index("s") * _NC + lax.axis_index("c")


_KCOL = _TCOL // _NW
_RB = 2


def _relayout_body(tT_hbm, tail_hbm, trm_hbm, stage0, stage1, trans0,
                   trans1, *sems):
    stages = [stage0, stage1]
    transs = [trans0, trans1]
    isems, osems = sems[:_RB], sems[_RB:]
    wid = _wid()
    iota32 = lax.iota(jnp.int32, 16) * _D

    def start_in(t, b):
        off = pl.multiple_of(t * 128, 128)
        pltpu.async_copy(tT_hbm.at[:, pl.ds(off, 128)], stages[b], isems[b])

    def drain_in(b):
        pltpu.make_async_copy(tT_hbm.at[:, pl.ds(0, 128)], stages[b],
                              isems[b]).wait()

    def start_out(t, b):
        pltpu.async_copy(transs[b], trm_hbm.at[pl.ds(t * 4096, 4096)],
                         osems[b])

    def drain_out(b):
        pltpu.make_async_copy(transs[b], trm_hbm.at[pl.ds(0, 4096)],
                              osems[b]).wait()

    iota = lax.iota(jnp.int32, 16)
    lvecs = [l0 * 16 + iota for l0 in range(8)]
    l32s = [(l0 * 16 + iota) * _D for l0 in range(8)]

    def transpose(b):
        @pl.loop(0, 16)
        def _diag(k):
            dk = (iota + k) & 15
            for d0 in (0, 16):
                dcol = d0 + dk
                for l0 in range(8):
                    v = plsc.load_gather(stages[b], [dcol, lvecs[l0]])
                    plsc.store_scatter(transs[b], [l32s[l0] + dcol], v)

    for b in range(_RB):
        start_in(wid + b * _NW, b)

    @pl.loop(0, _KCOL, step=_RB)
    def _col(g):
        for b in range(_RB):
            c = g + b
            t = wid + c * _NW
            drain_in(b)

            @pl.when(g > 0)
            def _():
                drain_out(b)

            transpose(b)
            start_out(t, b)

            @pl.when(c + _RB < _KCOL)
            def _():
                start_in(wid + (c + _RB) * _NW, b)

    for b in range(_RB):
        drain_out(b)

    @pl.when(wid < _TCOL - _KCOL * _NW)
    def _extra():
        t = wid + _KCOL * _NW
        off = pl.multiple_of(t * 128, 128)
        pltpu.sync_copy(tT_hbm.at[:, pl.ds(off, 128)], stages[0])
        transpose(0)
        pltpu.sync_copy(transs[0], trm_hbm.at[pl.ds(t * 4096, 4096)])

    @pl.when(wid == (_TCOL % _NW))
    def _tail():
        pltpu.sync_copy(tail_hbm,
                        trm_hbm.at[pl.ds(_TCOL * 128 * _D, 64 * _D)])


def _make_relayout():
    return functools.partial(
        pl.kernel,
        out_type=jax.ShapeDtypeStruct((_V * _D,), jnp.float32),
        mesh=_mesh(),
        scratch_types=(
            [pltpu.VMEM((_D, 128), jnp.float32)] * _RB
            + [pltpu.VMEM((128 * _D,), jnp.float32)] * _RB
            + [pltpu.SemaphoreType.DMA] * (2 * _RB)
        ),
        compiler_params=pltpu.CompilerParams(needs_layout_passes=False),
    )(_relayout_body)


_Q = _V // 4
_JPW = 4
_NPAIR = _HIST * _JPW


_LB = 2


def _lookup_body(x_hbm, t4_hbm, out_hbm, x_v, idx_b, rem_b, g_v, so_v, *sems):
    gsems, osems = sems[:_LB], sems[_LB:]
    wid = _wid()
    iota = lax.iota(jnp.int32, 16)
    iota50 = iota * 50
    pltpu.sync_copy(x_hbm.at[pl.ds(wid * _BPW, _BPW)], x_v)

    def fire(p, s):
        jj = p // _HIST
        h = p % _HIST
        for l0 in range(8):
            f_vec = iota50 + (jj * 6400 + l0 * 800 + h)
            xq = plsc.load_gather(x_v, [f_vec])
            idx_b[s, pl.ds(l0 * 16, 16)] = xq >> 2
            rem_b[s, pl.ds(l0 * 16, 16)] = (xq & 3) << 5
        pltpu.async_copy(t4_hbm.at[idx_b.at[s]], g_v.at[s], gsems[s])

    def drain_gather(s):
        pltpu.make_async_copy(t4_hbm.at[pl.ds(0, 128)], g_v.at[s],
                              gsems[s]).wait()

    def drain_out(s):
        pltpu.make_async_copy(so_v.at[s], out_hbm.at[0, :, pl.ds(0, 128)],
                              osems[s]).wait()

    def consume(p, s):
        jj = p // _HIST
        h = p % _HIST
        rows = [l0 * 16 + iota for l0 in range(8)]
        rembs = [rem_b[s, pl.ds(l0 * 16, 16)] for l0 in range(8)]
        @pl.loop(0, 16)
        def _diag(k):
            dk = (iota + k) & 15
            for d0 in (0, 16):
                dcol = d0 + dk
                for l0 in range(8):
                    v = plsc.load_gather(g_v.at[s],
                                         [rows[l0], rembs[l0] + dcol])
                    plsc.store_scatter(so_v.at[s], [dcol, rows[l0]], v)
        off = pl.multiple_of((wid * _JPW + jj) * 128, 128)
        pltpu.async_copy(so_v.at[s], out_hbm.at[h, :, pl.ds(off, 128)],
                         osems[s])

    for s in range(_LB):
        fire(s, s)

    @pl.loop(0, _NPAIR - _LB, step=_LB)
    def _pair(pp):
        for s in range(_LB):
            p = pp + s
            drain_gather(s)

            @pl.when(pp > 0)
            def _():
                drain_out(s)

            consume(p, s)
            fire(p + _LB, s)

    for s in range(_LB):
        drain_gather(s)
        drain_out(s)
        consume(_NPAIR - _LB + s, s)
    for s in range(_LB):
        drain_out(s)


def _make_lookup():
    return functools.partial(
        pl.kernel,
        out_type=jax.ShapeDtypeStruct((_HIST, _D, _BATCH), jnp.float32),
        mesh=_mesh(),
        scratch_types=(
            [pltpu.VMEM((_BPW,), jnp.int32),
             pltpu.VMEM((_LB, 128), jnp.int32),
             pltpu.VMEM((_LB, 128), jnp.int32),
             pltpu.VMEM((_LB, 128, 128), jnp.float32),
             pltpu.VMEM((_LB, _D, 128), jnp.float32)]
            + [pltpu.SemaphoreType.DMA] * (2 * _LB)
        ),
        compiler_params=pltpu.CompilerParams(needs_layout_passes=False),
    )(_lookup_body)


def kernel(x, table):
    xf = x.reshape(_B)
    tail = table[_TCOL * 128:].reshape(64 * _D)
    trm = _make_relayout()(table.T, tail)
    out = _make_lookup()(xf, trm.reshape(_Q, 128))
    return out.transpose(2, 0, 1)
